# baseline (speedup 1.0000x reference)
import functools

import jax
import jax.numpy as jnp
from jax import lax
from jax.experimental import pallas as pl
from jax.experimental.pallas import tpu as pltpu
from jax.experimental.pallas import tpu_sc as plsc

N = 10000
E = 320000
D = 128
H = 128
H2 = 256
L = 7
C = 40
EPS = 1e-7

R = 1000
GRID = N // R
CHUNK = 800
NCHUNK = E // CHUNK
CPT = 4
NEG = -1e30


def _ln(u, g, b):
    mu = jnp.mean(u, axis=-1, keepdims=True)
    var = jnp.var(u, axis=-1, keepdims=True)
    return (u - mu) / jnp.sqrt(var + 1e-5) * g + b


def _init_body(x_ref, w_ref, b_ref, a_ref, we_ref, be_ref,
               h_ref, hmax_ref, eamax_ref):
    i = pl.program_id(0)
    h = jnp.dot(x_ref[...], w_ref[...],
                preferred_element_type=jnp.float32) + b_ref[...]
    h_ref[...] = h

    @pl.when(i == 0)
    def _():
        hmax_ref[0, 0] = jnp.float32(NEG)
        aabs = jnp.maximum(jnp.max(a_ref[...]), -jnp.min(a_ref[...]))
        w = we_ref[...]
        ea = jnp.maximum(aabs * w, -aabs * w) + be_ref[...]
        eamax_ref[0, 0] = jnp.max(ea)

    hmax_ref[0, 0] = jnp.maximum(hmax_ref[0, 0], jnp.max(h))


def _init_call(x, node_W, node_b, a, edge_W, edge_b):
    smem = pl.BlockSpec((1, 1), lambda i: (0, 0), memory_space=pltpu.SMEM)
    return pl.pallas_call(
        _init_body,
        grid=(GRID,),
        in_specs=[
            pl.BlockSpec((R, D), lambda i: (i, 0)),
            pl.BlockSpec((D, H), lambda i: (0, 0)),
            pl.BlockSpec((1, H), lambda i: (0, 0)),
            pl.BlockSpec((E // 128, 128), lambda i: (0, 0)),
            pl.BlockSpec((1, H), lambda i: (0, 0)),
            pl.BlockSpec((1, H), lambda i: (0, 0)),
        ],
        out_specs=[pl.BlockSpec((R, H), lambda i: (i, 0)), smem, smem],
        out_shape=[
            jax.ShapeDtypeStruct((N, H), jnp.float32),
            jax.ShapeDtypeStruct((1, 1), jnp.float32),
            jax.ShapeDtypeStruct((1, 1), jnp.float32),
        ],
    )(x, node_W, node_b.reshape(1, H), a.reshape(-1, 128),
      edge_W, edge_b.reshape(1, H))


def _layer_body(has_res, h_ref, zin_ref, agg_ref, w1_ref, b1_ref, g1_ref,
                c1_ref, w2_ref, b2_ref, gn_ref, cn_ref,
                hn_ref, zn_ref, zmax_ref):
    i = pl.program_id(0)
    u = agg_ref[...] + zin_ref[...]
    u = jnp.dot(u, w1_ref[...], preferred_element_type=jnp.float32) + b1_ref[...]
    u = jnp.maximum(_ln(u, g1_ref[...], c1_ref[...]), 0.)
    u = jnp.dot(u, w2_ref[...], preferred_element_type=jnp.float32) + b2_ref[...]
    if has_res:
        hn = h_ref[...] + u
    else:
        hn = u
    hn_ref[...] = hn
    zn = jnp.maximum(_ln(hn, gn_ref[...], cn_ref[...]), 0.)
    zn_ref[...] = zn

    @pl.when(i == 0)
    def _():
        zmax_ref[0, 0] = jnp.float32(NEG)

    zmax_ref[0, 0] = jnp.maximum(zmax_ref[0, 0], jnp.max(zn))


def _layer_call(has_res, h, zin, agg, W1, b1, g1, c1, W2, b2, gn, cn):
    smem = pl.BlockSpec((1, 1), lambda i: (0, 0), memory_space=pltpu.SMEM)
    row = pl.BlockSpec((R, H), lambda i: (i, 0))
    const = lambda s: pl.BlockSpec(s, lambda i: (0, 0))
    return pl.pallas_call(
        functools.partial(_layer_body, has_res),
        grid=(GRID,),
        in_specs=[
            row, row, row,
            const((H, H2)), const((1, H2)), const((1, H2)), const((1, H2)),
            const((H2, H)), const((1, H)), const((1, H)), const((1, H)),
        ],
        out_specs=[row, row, smem],
        out_shape=[
            jax.ShapeDtypeStruct((N, H), jnp.float32),
            jax.ShapeDtypeStruct((N, H), jnp.float32),
            jax.ShapeDtypeStruct((1, 1), jnp.float32),
        ],
    )(h, zin, agg, W1, b1.reshape(1, H2), g1.reshape(1, H2),
      c1.reshape(1, H2), W2, b2.reshape(1, H), gn.reshape(1, H),
      cn.reshape(1, H))


def _head_body(z_ref, w_ref, b_ref, o_ref):
    logits = jnp.dot(z_ref[...], w_ref[...],
                     preferred_element_type=jnp.float32) + b_ref[...]
    mx = jnp.max(logits, axis=-1, keepdims=True)
    s = jnp.sum(jnp.exp(logits - mx), axis=-1, keepdims=True)
    o_ref[...] = logits - mx - jnp.log(s)


def _head_call(z, Wp, bp):
    return pl.pallas_call(
        _head_body,
        grid=(GRID,),
        in_specs=[
            pl.BlockSpec((R, H), lambda i: (i, 0)),
            pl.BlockSpec((H, 128), lambda i: (0, 0)),
            pl.BlockSpec((1, 128), lambda i: (0, 0)),
        ],
        out_specs=pl.BlockSpec((R, 128), lambda i: (i, 0)),
        out_shape=jax.ShapeDtypeStruct((N, 128), jnp.float32),
    )(z, Wp, bp)


def _conv_body(zT_hbm, src_hbm, dst_hbm, a_hbm, par_hbm, agg_hbm,
               z_v, den_v, num_v, src_v, dst_v, a_v, par_v):
    cid = lax.axis_index("c")
    sid = lax.axis_index("s")
    wid = sid * 2 + cid

    pltpu.sync_copy(zT_hbm.at[pl.ds(wid * CPT, CPT)], z_v)
    pltpu.sync_copy(par_hbm.at[wid], par_v)

    @plsc.parallel_loop(0, N // 16, 1, unroll=8)
    def zero_body(j):
        for c in range(CPT):
            den_v[c, pl.ds(j * 16, 16)] = jnp.zeros((16,), jnp.float32)
            num_v[c, pl.ds(j * 16, 16)] = jnp.zeros((16,), jnp.float32)

    wv = [par_v[c, :] for c in range(CPT)]
    bv = [par_v[CPT + c, :] for c in range(CPT)]
    tv = par_v[2 * CPT, :]
    Mv = par_v[2 * CPT + 1, :]

    c16 = [jnp.full((16,), c, jnp.int32) for c in range(CPT)]

    def chunk_body(g, carry):
        pltpu.sync_copy(src_hbm.at[pl.ds(g * CHUNK, CHUNK)], src_v)
        pltpu.sync_copy(dst_hbm.at[pl.ds(g * CHUNK, CHUNK)], dst_v)
        pltpu.sync_copy(a_hbm.at[pl.ds(g * CHUNK, CHUNK)], a_v)

        @plsc.parallel_loop(0, CHUNK // 16, 1, unroll=8)
        def edge_body(j):
            s = src_v[pl.ds(j * 16, 16)]
            d = dst_v[pl.ds(j * 16, 16)]
            av = a_v[pl.ds(j * 16, 16)]
            for c in range(CPT):
                zg = plsc.load_gather(z_v, [c16[c], s])
                m = jnp.maximum(zg + av * wv[c] + bv[c], 0.) + EPS
                ex = jnp.exp(tv * m - Mv)
                plsc.addupdate_scatter(den_v, [c16[c], d], ex)
                plsc.addupdate_scatter(num_v, [c16[c], d], ex * m)

        return carry

    lax.fori_loop(0, NCHUNK, chunk_body, 0)

    @plsc.parallel_loop(0, N // 16, 1, unroll=4)
    def div_body(j):
        sl = pl.ds(j * 16, 16)
        for c in range(CPT):
            num_v[c, sl] = num_v[c, sl] / (den_v[c, sl] + 1e-16)

    pltpu.sync_copy(num_v, agg_hbm.at[pl.ds(wid * CPT, CPT)])


@functools.cache
def _get_sc_conv():
    return pl.kernel(
        _conv_body,
        mesh=plsc.VectorSubcoreMesh(core_axis_name="c", subcore_axis_name="s"),
        out_type=jax.ShapeDtypeStruct((H, N), jnp.float32),
        scratch_types=[
            pltpu.VMEM((CPT, N), jnp.float32),
            pltpu.VMEM((CPT, N), jnp.float32),
            pltpu.VMEM((CPT, N), jnp.float32),
            pltpu.VMEM((CHUNK,), jnp.int32),
            pltpu.VMEM((CHUNK,), jnp.int32),
            pltpu.VMEM((CHUNK,), jnp.float32),
            pltpu.VMEM((2 * CPT + 2, 16), jnp.float32),
        ],
        compiler_params=pltpu.CompilerParams(needs_layout_passes=False),
    )


def _sc_conv(zT, src, dst, a, par):
    return _get_sc_conv()(zT, src, dst, a, par)


def kernel(x, edge_index, edge_attr, node_W, node_b, edge_W, edge_b, t,
           mlp_W1, mlp_b1, mlp_ln_g, mlp_ln_b, mlp_W2, mlp_b2, ln_g, ln_b,
           lin_W, lin_b):
    src = edge_index[0]
    dst = edge_index[1]
    a = edge_attr[:, 0]

    h0, hmax, eamax = _init_call(x, node_W, node_b, a, edge_W, edge_b)
    hmax = hmax[0, 0]
    eamax = eamax[0, 0]

    we_r = edge_W[0].reshape(32, CPT)
    be_r = edge_b.reshape(32, CPT)

    h = h0
    z = h0
    zmax = hmax
    for i in range(L):
        msgmax = jnp.maximum(zmax + eamax, 0.) + EPS
        M = jnp.maximum(t[i] * msgmax, t[i] * EPS)
        par = jnp.concatenate(
            [we_r, be_r,
             jnp.full((32, 1), 1., jnp.float32) * t[i],
             jnp.full((32, 1), 1., jnp.float32) * M], axis=1)
        par = jnp.broadcast_to(par[:, :, None], (32, 2 * CPT + 2, 16))
        aggT = _sc_conv(jnp.transpose(z), src, dst, a, par)
        agg = jnp.transpose(aggT)
        gn = ln_g[i + 1] if i < L - 1 else ln_g[0]
        cn = ln_b[i + 1] if i < L - 1 else ln_b[0]
        h, z, zmax = _layer_call(i > 0, h, z, agg, mlp_W1[i], mlp_b1[i],
                                 mlp_ln_g[i], mlp_ln_b[i], mlp_W2[i],
                                 mlp_b2[i], gn, cn)
        zmax = zmax[0, 0]

    Wp = jnp.pad(lin_W, ((0, 0), (0, 128 - C)))
    bp = jnp.pad(lin_b, (0, 128 - C), constant_values=NEG).reshape(1, 128)
    out = _head_call(z, Wp, bp)
    return out[:, :C]


# speedup vs baseline: 2.3343x; 2.3343x over previous
import functools

import jax
import jax.numpy as jnp
from jax import lax
from jax.experimental import pallas as pl
from jax.experimental.pallas import tpu as pltpu
from jax.experimental.pallas import tpu_sc as plsc

N = 10000
E = 320000
D = 128
H = 128
H2 = 256
L = 7
C = 40
EPS = 1e-7

R = 1000
GRID = N // R
CHUNK = 1600
NCHUNK = E // CHUNK
CPT = 4
NEG = -1e30


def _ln(u, g, b):
    mu = jnp.mean(u, axis=-1, keepdims=True)
    var = jnp.var(u, axis=-1, keepdims=True)
    return (u - mu) / jnp.sqrt(var + 1e-5) * g + b


def _init_body(x_ref, w_ref, b_ref, a_ref, we_ref, be_ref,
               h_ref, hmax_ref, eamax_ref):
    i = pl.program_id(0)
    h = jnp.dot(x_ref[...], w_ref[...],
                preferred_element_type=jnp.float32) + b_ref[...]
    h_ref[...] = h

    @pl.when(i == 0)
    def _():
        hmax_ref[0, 0] = jnp.float32(NEG)
        aabs = jnp.maximum(jnp.max(a_ref[...]), -jnp.min(a_ref[...]))
        w = we_ref[...]
        ea = jnp.maximum(aabs * w, -aabs * w) + be_ref[...]
        eamax_ref[0, 0] = jnp.max(ea)

    hmax_ref[0, 0] = jnp.maximum(hmax_ref[0, 0], jnp.max(h))


def _init_call(x, node_W, node_b, a, edge_W, edge_b):
    smem = pl.BlockSpec((1, 1), lambda i: (0, 0), memory_space=pltpu.SMEM)
    return pl.pallas_call(
        _init_body,
        grid=(GRID,),
        in_specs=[
            pl.BlockSpec((R, D), lambda i: (i, 0)),
            pl.BlockSpec((D, H), lambda i: (0, 0)),
            pl.BlockSpec((1, H), lambda i: (0, 0)),
            pl.BlockSpec((E // 128, 128), lambda i: (0, 0)),
            pl.BlockSpec((1, H), lambda i: (0, 0)),
            pl.BlockSpec((1, H), lambda i: (0, 0)),
        ],
        out_specs=[pl.BlockSpec((R, H), lambda i: (i, 0)), smem, smem],
        out_shape=[
            jax.ShapeDtypeStruct((N, H), jnp.float32),
            jax.ShapeDtypeStruct((1, 1), jnp.float32),
            jax.ShapeDtypeStruct((1, 1), jnp.float32),
        ],
    )(x, node_W, node_b.reshape(1, H), a.reshape(-1, 128),
      edge_W, edge_b.reshape(1, H))


def _layer_body(has_res, h_ref, zin_ref, agg_ref, w1_ref, b1_ref, g1_ref,
                c1_ref, w2_ref, b2_ref, gn_ref, cn_ref,
                hn_ref, zn_ref, zmax_ref):
    i = pl.program_id(0)
    u = agg_ref[...] + zin_ref[...]
    u = jnp.dot(u, w1_ref[...], preferred_element_type=jnp.float32) + b1_ref[...]
    u = jnp.maximum(_ln(u, g1_ref[...], c1_ref[...]), 0.)
    u = jnp.dot(u, w2_ref[...], preferred_element_type=jnp.float32) + b2_ref[...]
    if has_res:
        hn = h_ref[...] + u
    else:
        hn = u
    hn_ref[...] = hn
    zn = jnp.maximum(_ln(hn, gn_ref[...], cn_ref[...]), 0.)
    zn_ref[...] = zn

    @pl.when(i == 0)
    def _():
        zmax_ref[0, 0] = jnp.float32(NEG)

    zmax_ref[0, 0] = jnp.maximum(zmax_ref[0, 0], jnp.max(zn))


def _layer_call(has_res, h, zin, agg, W1, b1, g1, c1, W2, b2, gn, cn):
    smem = pl.BlockSpec((1, 1), lambda i: (0, 0), memory_space=pltpu.SMEM)
    row = pl.BlockSpec((R, H), lambda i: (i, 0))
    const = lambda s: pl.BlockSpec(s, lambda i: (0, 0))
    return pl.pallas_call(
        functools.partial(_layer_body, has_res),
        grid=(GRID,),
        in_specs=[
            row, row, row,
            const((H, H2)), const((1, H2)), const((1, H2)), const((1, H2)),
            const((H2, H)), const((1, H)), const((1, H)), const((1, H)),
        ],
        out_specs=[row, row, smem],
        out_shape=[
            jax.ShapeDtypeStruct((N, H), jnp.float32),
            jax.ShapeDtypeStruct((N, H), jnp.float32),
            jax.ShapeDtypeStruct((1, 1), jnp.float32),
        ],
    )(h, zin, agg, W1, b1.reshape(1, H2), g1.reshape(1, H2),
      c1.reshape(1, H2), W2, b2.reshape(1, H), gn.reshape(1, H),
      cn.reshape(1, H))


def _head_body(z_ref, w_ref, b_ref, o_ref):
    logits = jnp.dot(z_ref[...], w_ref[...],
                     preferred_element_type=jnp.float32) + b_ref[...]
    mx = jnp.max(logits, axis=-1, keepdims=True)
    s = jnp.sum(jnp.exp(logits - mx), axis=-1, keepdims=True)
    o_ref[...] = logits - mx - jnp.log(s)


def _head_call(z, Wp, bp):
    return pl.pallas_call(
        _head_body,
        grid=(GRID,),
        in_specs=[
            pl.BlockSpec((R, H), lambda i: (i, 0)),
            pl.BlockSpec((H, 128), lambda i: (0, 0)),
            pl.BlockSpec((1, 128), lambda i: (0, 0)),
        ],
        out_specs=pl.BlockSpec((R, 128), lambda i: (i, 0)),
        out_shape=jax.ShapeDtypeStruct((N, 128), jnp.float32),
    )(z, Wp, bp)


def _conv_body(zT_hbm, src_hbm, dst_hbm, a_hbm, par_hbm, agg_hbm,
               z_v, den_v, num_v, src_v, dst_v, a_v, par_v):
    cid = lax.axis_index("c")
    sid = lax.axis_index("s")
    wid = sid * 2 + cid

    pltpu.sync_copy(zT_hbm.at[pl.ds(wid * CPT, CPT)], z_v)
    pltpu.sync_copy(par_hbm.at[wid], par_v)

    @plsc.parallel_loop(0, N // 16, 1, unroll=8)
    def zero_body(j):
        for c in range(CPT):
            den_v[c, pl.ds(j * 16, 16)] = jnp.zeros((16,), jnp.float32)
            num_v[c, pl.ds(j * 16, 16)] = jnp.zeros((16,), jnp.float32)

    wv = [par_v[c, :] for c in range(CPT)]
    bv = [par_v[CPT + c, :] for c in range(CPT)]
    tv = par_v[2 * CPT, :]
    Mv = par_v[2 * CPT + 1, :]

    c16 = [jnp.full((16,), c, jnp.int32) for c in range(CPT)]

    def chunk_body(g, carry):
        pltpu.sync_copy(src_hbm.at[pl.ds(g * CHUNK, CHUNK)], src_v)
        pltpu.sync_copy(dst_hbm.at[pl.ds(g * CHUNK, CHUNK)], dst_v)
        pltpu.sync_copy(a_hbm.at[pl.ds(g * CHUNK, CHUNK)], a_v)

        @plsc.parallel_loop(0, CHUNK // 16, 1, unroll=4)
        def edge_body(j):
            s = src_v[pl.ds(j * 16, 16)]
            d = dst_v[pl.ds(j * 16, 16)]
            av = a_v[pl.ds(j * 16, 16)]
            for c in range(CPT):
                zg = plsc.load_gather(z_v, [c16[c], s])
                m = jnp.maximum(zg + av * wv[c] + bv[c], 0.) + EPS
                ex = jnp.exp(tv * m - Mv)
                plsc.addupdate_scatter(den_v, [c16[c], d], ex)
                plsc.addupdate_scatter(num_v, [c16[c], d], ex * m)

        return carry

    lax.fori_loop(0, NCHUNK, chunk_body, 0)

    @plsc.parallel_loop(0, N // 16, 1, unroll=4)
    def div_body(j):
        sl = pl.ds(j * 16, 16)
        for c in range(CPT):
            num_v[c, sl] = num_v[c, sl] / (den_v[c, sl] + 1e-16)

    pltpu.sync_copy(num_v, agg_hbm.at[pl.ds(wid * CPT, CPT)])


@functools.cache
def _get_sc_conv():
    return pl.kernel(
        _conv_body,
        mesh=plsc.VectorSubcoreMesh(core_axis_name="c", subcore_axis_name="s"),
        out_type=jax.ShapeDtypeStruct((H, N), jnp.float32),
        scratch_types=[
            pltpu.VMEM((CPT, N), jnp.float32),
            pltpu.VMEM((CPT, N), jnp.float32),
            pltpu.VMEM((CPT, N), jnp.float32),
            pltpu.VMEM((CHUNK,), jnp.int32),
            pltpu.VMEM((CHUNK,), jnp.int32),
            pltpu.VMEM((CHUNK,), jnp.float32),
            pltpu.VMEM((2 * CPT + 2, 16), jnp.float32),
        ],
        compiler_params=pltpu.CompilerParams(needs_layout_passes=False),
    )


def _sc_conv(zT, src, dst, a, par):
    return _get_sc_conv()(zT, src, dst, a, par)


def kernel(x, edge_index, edge_attr, node_W, node_b, edge_W, edge_b, t,
           mlp_W1, mlp_b1, mlp_ln_g, mlp_ln_b, mlp_W2, mlp_b2, ln_g, ln_b,
           lin_W, lin_b):
    src = edge_index[0]
    dst = edge_index[1]
    a = edge_attr[:, 0]

    h0, hmax, eamax = _init_call(x, node_W, node_b, a, edge_W, edge_b)
    hmax = hmax[0, 0]
    eamax = eamax[0, 0]

    we_r = edge_W[0].reshape(32, CPT)
    be_r = edge_b.reshape(32, CPT)

    h = h0
    z = h0
    zmax = hmax
    for i in range(L):
        msgmax = jnp.maximum(zmax + eamax, 0.) + EPS
        M = jnp.maximum(t[i] * msgmax, t[i] * EPS)
        par = jnp.concatenate(
            [we_r, be_r,
             jnp.full((32, 1), 1., jnp.float32) * t[i],
             jnp.full((32, 1), 1., jnp.float32) * M], axis=1)
        par = jnp.broadcast_to(par[:, :, None], (32, 2 * CPT + 2, 16))
        aggT = _sc_conv(jnp.transpose(z), src, dst, a, par)
        agg = jnp.transpose(aggT)
        gn = ln_g[i + 1] if i < L - 1 else ln_g[0]
        cn = ln_b[i + 1] if i < L - 1 else ln_b[0]
        h, z, zmax = _layer_call(i > 0, h, z, agg, mlp_W1[i], mlp_b1[i],
                                 mlp_ln_g[i], mlp_ln_b[i], mlp_W2[i],
                                 mlp_b2[i], gn, cn)
        zmax = zmax[0, 0]

    Wp = jnp.pad(lin_W, ((0, 0), (0, 128 - C)))
    bp = jnp.pad(lin_b, (0, 128 - C), constant_values=NEG).reshape(1, 128)
    out = _head_call(z, Wp, bp)
    return out[:, :C]


# speedup vs baseline: 2.9831x; 1.2780x over previous
import functools

import jax
import jax.numpy as jnp
from jax import lax
from jax.experimental import pallas as pl
from jax.experimental.pallas import tpu as pltpu
from jax.experimental.pallas import tpu_sc as plsc

N = 10000
E = 320000
D = 128
H = 128
H2 = 256
L = 7
C = 40
EPS = 1e-7

R = 1000
GRID = N // R
CHUNK = 1280
NCHUNK = E // CHUNK
CPT = 4
NEG = -1e30


def _ln(u, g, b):
    mu = jnp.mean(u, axis=-1, keepdims=True)
    var = jnp.var(u, axis=-1, keepdims=True)
    return (u - mu) / jnp.sqrt(var + 1e-5) * g + b


def _init_body(x_ref, w_ref, b_ref, a_ref, we_ref, be_ref,
               h_ref, hmax_ref, eamax_ref):
    i = pl.program_id(0)
    h = jnp.dot(x_ref[...], w_ref[...],
                preferred_element_type=jnp.float32) + b_ref[...]
    h_ref[...] = h

    @pl.when(i == 0)
    def _():
        hmax_ref[0, 0] = jnp.float32(NEG)
        aabs = jnp.maximum(jnp.max(a_ref[...]), -jnp.min(a_ref[...]))
        w = we_ref[...]
        ea = jnp.maximum(aabs * w, -aabs * w) + be_ref[...]
        eamax_ref[0, 0] = jnp.max(ea)

    hmax_ref[0, 0] = jnp.maximum(hmax_ref[0, 0], jnp.max(h))


def _init_call(x, node_W, node_b, a, edge_W, edge_b):
    smem = pl.BlockSpec((1, 1), lambda i: (0, 0), memory_space=pltpu.SMEM)
    return pl.pallas_call(
        _init_body,
        grid=(GRID,),
        in_specs=[
            pl.BlockSpec((R, D), lambda i: (i, 0)),
            pl.BlockSpec((D, H), lambda i: (0, 0)),
            pl.BlockSpec((1, H), lambda i: (0, 0)),
            pl.BlockSpec((E // 128, 128), lambda i: (0, 0)),
            pl.BlockSpec((1, H), lambda i: (0, 0)),
            pl.BlockSpec((1, H), lambda i: (0, 0)),
        ],
        out_specs=[pl.BlockSpec((R, H), lambda i: (i, 0)), smem, smem],
        out_shape=[
            jax.ShapeDtypeStruct((N, H), jnp.float32),
            jax.ShapeDtypeStruct((1, 1), jnp.float32),
            jax.ShapeDtypeStruct((1, 1), jnp.float32),
        ],
    )(x, node_W, node_b.reshape(1, H), a.reshape(-1, 128),
      edge_W, edge_b.reshape(1, H))


def _layer_body(has_res, h_ref, zin_ref, agg_ref, w1_ref, b1_ref, g1_ref,
                c1_ref, w2_ref, b2_ref, gn_ref, cn_ref,
                hn_ref, zn_ref, zmax_ref):
    i = pl.program_id(0)
    u = agg_ref[...] + zin_ref[...]
    u = jnp.dot(u, w1_ref[...], preferred_element_type=jnp.float32) + b1_ref[...]
    u = jnp.maximum(_ln(u, g1_ref[...], c1_ref[...]), 0.)
    u = jnp.dot(u, w2_ref[...], preferred_element_type=jnp.float32) + b2_ref[...]
    if has_res:
        hn = h_ref[...] + u
    else:
        hn = u
    hn_ref[...] = hn
    zn = jnp.maximum(_ln(hn, gn_ref[...], cn_ref[...]), 0.)
    zn_ref[...] = zn

    @pl.when(i == 0)
    def _():
        zmax_ref[0, 0] = jnp.float32(NEG)

    zmax_ref[0, 0] = jnp.maximum(zmax_ref[0, 0], jnp.max(zn))


def _layer_call(has_res, h, zin, agg, W1, b1, g1, c1, W2, b2, gn, cn):
    smem = pl.BlockSpec((1, 1), lambda i: (0, 0), memory_space=pltpu.SMEM)
    row = pl.BlockSpec((R, H), lambda i: (i, 0))
    const = lambda s: pl.BlockSpec(s, lambda i: (0, 0))
    return pl.pallas_call(
        functools.partial(_layer_body, has_res),
        grid=(GRID,),
        in_specs=[
            row, row, row,
            const((H, H2)), const((1, H2)), const((1, H2)), const((1, H2)),
            const((H2, H)), const((1, H)), const((1, H)), const((1, H)),
        ],
        out_specs=[row, row, smem],
        out_shape=[
            jax.ShapeDtypeStruct((N, H), jnp.float32),
            jax.ShapeDtypeStruct((N, H), jnp.float32),
            jax.ShapeDtypeStruct((1, 1), jnp.float32),
        ],
    )(h, zin, agg, W1, b1.reshape(1, H2), g1.reshape(1, H2),
      c1.reshape(1, H2), W2, b2.reshape(1, H), gn.reshape(1, H),
      cn.reshape(1, H))


def _head_body(z_ref, w_ref, b_ref, o_ref):
    logits = jnp.dot(z_ref[...], w_ref[...],
                     preferred_element_type=jnp.float32) + b_ref[...]
    mx = jnp.max(logits, axis=-1, keepdims=True)
    s = jnp.sum(jnp.exp(logits - mx), axis=-1, keepdims=True)
    o_ref[...] = logits - mx - jnp.log(s)


def _head_call(z, Wp, bp):
    return pl.pallas_call(
        _head_body,
        grid=(GRID,),
        in_specs=[
            pl.BlockSpec((R, H), lambda i: (i, 0)),
            pl.BlockSpec((H, 128), lambda i: (0, 0)),
            pl.BlockSpec((1, 128), lambda i: (0, 0)),
        ],
        out_specs=pl.BlockSpec((R, 128), lambda i: (i, 0)),
        out_shape=jax.ShapeDtypeStruct((N, 128), jnp.float32),
    )(z, Wp, bp)


def _conv_body(zT_hbm, eidx_hbm, par_hbm, agg_hbm,
               z_v, den_v, num_v, e_v, par_v):
    cid = lax.axis_index("c")
    sid = lax.axis_index("s")
    wid = sid * 2 + cid

    pltpu.sync_copy(zT_hbm.at[pl.ds(wid * CPT, CPT)], z_v)
    pltpu.sync_copy(par_hbm.at[wid], par_v)

    @plsc.parallel_loop(0, N // 16, 1, unroll=8)
    def zero_body(j):
        for c in range(CPT):
            den_v[c, pl.ds(j * 16, 16)] = jnp.zeros((16,), jnp.float32)
            num_v[c, pl.ds(j * 16, 16)] = jnp.zeros((16,), jnp.float32)

    wv = [par_v[c, :] for c in range(CPT)]
    bv = [par_v[CPT + c, :] for c in range(CPT)]
    tv = par_v[2 * CPT, :]
    Mv = par_v[2 * CPT + 1, :]

    c16 = [jnp.full((16,), c, jnp.int32) for c in range(CPT)]

    def chunk_body(g, carry):
        pltpu.sync_copy(eidx_hbm.at[:, pl.ds(g * CHUNK, CHUNK)], e_v)

        @plsc.parallel_loop(0, CHUNK // 16, 1, unroll=4)
        def edge_body(j):
            s = e_v[0, pl.ds(j * 16, 16)]
            d = e_v[1, pl.ds(j * 16, 16)]
            av = plsc.bitcast(e_v[2, pl.ds(j * 16, 16)], jnp.float32)
            for c in range(CPT):
                zg = plsc.load_gather(z_v, [c16[c], s])
                m = jnp.maximum(zg + av * wv[c] + bv[c], 0.) + EPS
                ex = jnp.exp(tv * m - Mv)
                plsc.addupdate_scatter(den_v, [c16[c], d], ex)
                plsc.addupdate_scatter(num_v, [c16[c], d], ex * m)

        return carry

    lax.fori_loop(0, NCHUNK, chunk_body, 0)

    @plsc.parallel_loop(0, N // 16, 1, unroll=4)
    def div_body(j):
        sl = pl.ds(j * 16, 16)
        for c in range(CPT):
            num_v[c, sl] = num_v[c, sl] / (den_v[c, sl] + 1e-16)

    pltpu.sync_copy(num_v, agg_hbm.at[pl.ds(wid * CPT, CPT)])


@functools.cache
def _get_sc_conv():
    return pl.kernel(
        _conv_body,
        mesh=plsc.VectorSubcoreMesh(core_axis_name="c", subcore_axis_name="s"),
        out_type=jax.ShapeDtypeStruct((H, N), jnp.float32),
        scratch_types=[
            pltpu.VMEM((CPT, N), jnp.float32),
            pltpu.VMEM((CPT, N), jnp.float32),
            pltpu.VMEM((CPT, N), jnp.float32),
            pltpu.VMEM((3, CHUNK), jnp.int32),
            pltpu.VMEM((2 * CPT + 2, 16), jnp.float32),
        ],
        compiler_params=pltpu.CompilerParams(needs_layout_passes=False),
    )


def _sc_conv(zT, eidx, par):
    return _get_sc_conv()(zT, eidx, par)


def kernel(x, edge_index, edge_attr, node_W, node_b, edge_W, edge_b, t,
           mlp_W1, mlp_b1, mlp_ln_g, mlp_ln_b, mlp_W2, mlp_b2, ln_g, ln_b,
           lin_W, lin_b):
    a = edge_attr[:, 0]
    eidx = jnp.concatenate(
        [edge_index, lax.bitcast_convert_type(a, jnp.int32)[None]], axis=0)

    h0, hmax, eamax = _init_call(x, node_W, node_b, a, edge_W, edge_b)
    hmax = hmax[0, 0]
    eamax = eamax[0, 0]

    we_r = edge_W[0].reshape(32, CPT)
    be_r = edge_b.reshape(32, CPT)

    h = h0
    z = h0
    zmax = hmax
    for i in range(L):
        msgmax = jnp.maximum(zmax + eamax, 0.) + EPS
        M = jnp.maximum(t[i] * msgmax, t[i] * EPS)
        par = jnp.concatenate(
            [we_r, be_r,
             jnp.full((32, 1), 1., jnp.float32) * t[i],
             jnp.full((32, 1), 1., jnp.float32) * M], axis=1)
        par = jnp.broadcast_to(par[:, :, None], (32, 2 * CPT + 2, 16))
        aggT = _sc_conv(jnp.transpose(z), eidx, par)
        agg = jnp.transpose(aggT)
        gn = ln_g[i + 1] if i < L - 1 else ln_g[0]
        cn = ln_b[i + 1] if i < L - 1 else ln_b[0]
        h, z, zmax = _layer_call(i > 0, h, z, agg, mlp_W1[i], mlp_b1[i],
                                 mlp_ln_g[i], mlp_ln_b[i], mlp_W2[i],
                                 mlp_b2[i], gn, cn)
        zmax = zmax[0, 0]

    Wp = jnp.pad(lin_W, ((0, 0), (0, 128 - C)))
    bp = jnp.pad(lin_b, (0, 128 - C), constant_values=NEG).reshape(1, 128)
    out = _head_call(z, Wp, bp)
    return out[:, :C]


# speedup vs baseline: 3.4004x; 1.1399x over previous
import functools

import jax
import jax.numpy as jnp
from jax import lax
from jax.experimental import pallas as pl
from jax.experimental.pallas import tpu as pltpu
from jax.experimental.pallas import tpu_sc as plsc

N = 10000
E = 320000
D = 128
H = 128
H2 = 256
L = 7
C = 40
EPS = 1e-7

R = 1000
GRID = N // R
CHUNK = 2560
NCHUNK = E // CHUNK
CPT = 4
PARW = (2 * CPT + 2) * 16
NEG = -1e30


def _ln(u, g, b):
    mu = jnp.mean(u, axis=-1, keepdims=True)
    var = jnp.var(u, axis=-1, keepdims=True)
    return (u - mu) / jnp.sqrt(var + 1e-5) * g + b


def _init_body(x_ref, w_ref, b_ref, a_ref, we_ref, be_ref,
               h_ref, hmax_ref, eamax_ref):
    i = pl.program_id(0)
    h = jnp.dot(x_ref[...], w_ref[...],
                preferred_element_type=jnp.float32) + b_ref[...]
    h_ref[...] = h

    @pl.when(i == 0)
    def _():
        hmax_ref[0, 0] = jnp.float32(NEG)
        aabs = jnp.maximum(jnp.max(a_ref[...]), -jnp.min(a_ref[...]))
        w = we_ref[...]
        ea = jnp.maximum(aabs * w, -aabs * w) + be_ref[...]
        eamax_ref[0, 0] = jnp.max(ea)

    hmax_ref[0, 0] = jnp.maximum(hmax_ref[0, 0], jnp.max(h))


def _init_call(x, node_W, node_b, a, edge_W, edge_b):
    smem = pl.BlockSpec((1, 1), lambda i: (0, 0), memory_space=pltpu.SMEM)
    return pl.pallas_call(
        _init_body,
        grid=(GRID,),
        in_specs=[
            pl.BlockSpec((R, D), lambda i: (i, 0)),
            pl.BlockSpec((D, H), lambda i: (0, 0)),
            pl.BlockSpec((1, H), lambda i: (0, 0)),
            pl.BlockSpec((E // 128, 128), lambda i: (0, 0)),
            pl.BlockSpec((1, H), lambda i: (0, 0)),
            pl.BlockSpec((1, H), lambda i: (0, 0)),
        ],
        out_specs=[pl.BlockSpec((R, H), lambda i: (i, 0)), smem, smem],
        out_shape=[
            jax.ShapeDtypeStruct((N, H), jnp.float32),
            jax.ShapeDtypeStruct((1, 1), jnp.float32),
            jax.ShapeDtypeStruct((1, 1), jnp.float32),
        ],
    )(x, node_W, node_b.reshape(1, H), a.reshape(-1, 128),
      edge_W, edge_b.reshape(1, H))


def _layer_body(has_res, h_ref, zin_ref, agg_ref, w1_ref, b1_ref, g1_ref,
                c1_ref, w2_ref, b2_ref, gn_ref, cn_ref,
                hn_ref, zn_ref, zmax_ref):
    i = pl.program_id(0)
    u = agg_ref[...] + zin_ref[...]
    u = jnp.dot(u, w1_ref[...], preferred_element_type=jnp.float32) + b1_ref[...]
    u = jnp.maximum(_ln(u, g1_ref[...], c1_ref[...]), 0.)
    u = jnp.dot(u, w2_ref[...], preferred_element_type=jnp.float32) + b2_ref[...]
    if has_res:
        hn = h_ref[...] + u
    else:
        hn = u
    hn_ref[...] = hn
    zn = jnp.maximum(_ln(hn, gn_ref[...], cn_ref[...]), 0.)
    zn_ref[...] = zn

    @pl.when(i == 0)
    def _():
        zmax_ref[0, 0] = jnp.float32(NEG)

    zmax_ref[0, 0] = jnp.maximum(zmax_ref[0, 0], jnp.max(zn))


def _layer_call(has_res, h, zin, agg, W1, b1, g1, c1, W2, b2, gn, cn):
    smem = pl.BlockSpec((1, 1), lambda i: (0, 0), memory_space=pltpu.SMEM)
    row = pl.BlockSpec((R, H), lambda i: (i, 0))
    const = lambda s: pl.BlockSpec(s, lambda i: (0, 0))
    return pl.pallas_call(
        functools.partial(_layer_body, has_res),
        grid=(GRID,),
        in_specs=[
            row, row, row,
            const((H, H2)), const((1, H2)), const((1, H2)), const((1, H2)),
            const((H2, H)), const((1, H)), const((1, H)), const((1, H)),
        ],
        out_specs=[row, row, smem],
        out_shape=[
            jax.ShapeDtypeStruct((N, H), jnp.float32),
            jax.ShapeDtypeStruct((N, H), jnp.float32),
            jax.ShapeDtypeStruct((1, 1), jnp.float32),
        ],
    )(h, zin, agg, W1, b1.reshape(1, H2), g1.reshape(1, H2),
      c1.reshape(1, H2), W2, b2.reshape(1, H), gn.reshape(1, H),
      cn.reshape(1, H))


def _head_body(z_ref, w_ref, b_ref, o_ref):
    logits = jnp.dot(z_ref[...], w_ref[...],
                     preferred_element_type=jnp.float32) + b_ref[...]
    mx = jnp.max(logits, axis=-1, keepdims=True)
    s = jnp.sum(jnp.exp(logits - mx), axis=-1, keepdims=True)
    o_ref[...] = logits - mx - jnp.log(s)


def _head_call(z, Wp, bp):
    return pl.pallas_call(
        _head_body,
        grid=(GRID,),
        in_specs=[
            pl.BlockSpec((R, H), lambda i: (i, 0)),
            pl.BlockSpec((H, 128), lambda i: (0, 0)),
            pl.BlockSpec((1, 128), lambda i: (0, 0)),
        ],
        out_specs=pl.BlockSpec((R, 128), lambda i: (i, 0)),
        out_shape=jax.ShapeDtypeStruct((N, 128), jnp.float32),
    )(z, Wp, bp)


def _conv_body(zT_hbm, eidx_hbm, par_hbm, agg_hbm,
               z_v, den_v, num_v, e_v, par_v):
    cid = lax.axis_index("c")
    sid = lax.axis_index("s")
    wid = sid * 2 + cid

    pltpu.sync_copy(zT_hbm.at[pl.ds(wid * (CPT * N), CPT * N)], z_v)
    pltpu.sync_copy(par_hbm.at[pl.ds(wid * PARW, PARW)], par_v)

    @plsc.parallel_loop(0, N // 16, 1, unroll=8)
    def zero_body(j):
        for c in range(CPT):
            den_v[pl.ds(c * N + j * 16, 16)] = jnp.zeros((16,), jnp.float32)
            num_v[pl.ds(c * N + j * 16, 16)] = jnp.zeros((16,), jnp.float32)

    wv = [par_v[pl.ds(c * 16, 16)] for c in range(CPT)]
    bv = [par_v[pl.ds((CPT + c) * 16, 16)] for c in range(CPT)]
    tv = par_v[pl.ds(2 * CPT * 16, 16)]
    Mv = par_v[pl.ds((2 * CPT + 1) * 16, 16)]

    cN = [jnp.full((16,), c * N, jnp.int32) for c in range(CPT)]

    def chunk_body(g, carry):
        pltpu.sync_copy(eidx_hbm.at[pl.ds(g * (3 * CHUNK), 3 * CHUNK)], e_v)

        @plsc.parallel_loop(0, CHUNK // 16, 1, unroll=4)
        def edge_body(j):
            s = e_v[pl.ds(j * 16, 16)]
            d = e_v[pl.ds(CHUNK + j * 16, 16)]
            av = plsc.bitcast(e_v[pl.ds(2 * CHUNK + j * 16, 16)],
                              jnp.float32)
            for c in range(CPT):
                zg = plsc.load_gather(z_v, [s + cN[c]])
                m = jnp.maximum(zg + av * wv[c] + bv[c], 0.) + EPS
                ex = jnp.exp(tv * m - Mv)
                dc = d + cN[c]
                plsc.addupdate_scatter(den_v, [dc], ex)
                plsc.addupdate_scatter(num_v, [dc], ex * m)

        return carry

    lax.fori_loop(0, NCHUNK, chunk_body, 0)

    @plsc.parallel_loop(0, N // 16, 1, unroll=4)
    def div_body(j):
        for c in range(CPT):
            sl = pl.ds(c * N + j * 16, 16)
            num_v[sl] = num_v[sl] / (den_v[sl] + 1e-16)

    pltpu.sync_copy(num_v, agg_hbm.at[pl.ds(wid * (CPT * N), CPT * N)])


@functools.cache
def _get_sc_conv():
    return pl.kernel(
        _conv_body,
        mesh=plsc.VectorSubcoreMesh(core_axis_name="c", subcore_axis_name="s"),
        out_type=jax.ShapeDtypeStruct((H * N,), jnp.float32),
        scratch_types=[
            pltpu.VMEM((CPT * N,), jnp.float32),
            pltpu.VMEM((CPT * N,), jnp.float32),
            pltpu.VMEM((CPT * N,), jnp.float32),
            pltpu.VMEM((3 * CHUNK,), jnp.int32),
            pltpu.VMEM((PARW,), jnp.float32),
        ],
        compiler_params=pltpu.CompilerParams(needs_layout_passes=False),
    )


def _sc_conv(zT, eidx, par):
    return _get_sc_conv()(zT, eidx, par)


def kernel(x, edge_index, edge_attr, node_W, node_b, edge_W, edge_b, t,
           mlp_W1, mlp_b1, mlp_ln_g, mlp_ln_b, mlp_W2, mlp_b2, ln_g, ln_b,
           lin_W, lin_b):
    a = edge_attr[:, 0]
    eidx = jnp.concatenate(
        [edge_index, lax.bitcast_convert_type(a, jnp.int32)[None]], axis=0)
    eidx = jnp.transpose(eidx.reshape(3, NCHUNK, CHUNK),
                         (1, 0, 2)).reshape(-1)

    h0, hmax, eamax = _init_call(x, node_W, node_b, a, edge_W, edge_b)
    hmax = hmax[0, 0]
    eamax = eamax[0, 0]

    we_r = edge_W[0].reshape(32, CPT)
    be_r = edge_b.reshape(32, CPT)

    h = h0
    z = h0
    zmax = hmax
    for i in range(L):
        msgmax = jnp.maximum(zmax + eamax, 0.) + EPS
        M = jnp.maximum(t[i] * msgmax, t[i] * EPS)
        par = jnp.concatenate(
            [we_r, be_r,
             jnp.full((32, 1), 1., jnp.float32) * t[i],
             jnp.full((32, 1), 1., jnp.float32) * M], axis=1)
        par = jnp.broadcast_to(par[:, :, None],
                               (32, 2 * CPT + 2, 16)).reshape(-1)
        aggT = _sc_conv(jnp.transpose(z).reshape(-1), eidx, par)
        agg = jnp.transpose(aggT.reshape(H, N))
        gn = ln_g[i + 1] if i < L - 1 else ln_g[0]
        cn = ln_b[i + 1] if i < L - 1 else ln_b[0]
        h, z, zmax = _layer_call(i > 0, h, z, agg, mlp_W1[i], mlp_b1[i],
                                 mlp_ln_g[i], mlp_ln_b[i], mlp_W2[i],
                                 mlp_b2[i], gn, cn)
        zmax = zmax[0, 0]

    Wp = jnp.pad(lin_W, ((0, 0), (0, 128 - C)))
    bp = jnp.pad(lin_b, (0, 128 - C), constant_values=NEG).reshape(1, 128)
    out = _head_call(z, Wp, bp)
    return out[:, :C]


# speedup vs baseline: 3.5127x; 1.0330x over previous
import functools

import jax
import jax.numpy as jnp
from jax import lax
from jax.experimental import pallas as pl
from jax.experimental.pallas import tpu as pltpu
from jax.experimental.pallas import tpu_sc as plsc

N = 10000
E = 320000
D = 128
H = 128
H2 = 256
L = 7
C = 40
EPS = 1e-7

R = 1000
GRID = N // R
CHUNK = 2560
NCHUNK = E // CHUNK
CPT = 4
PARW = (2 * CPT + 2) * 16
NEG = -1e30


def _ln(u, g, b):
    mu = jnp.mean(u, axis=-1, keepdims=True)
    var = jnp.var(u, axis=-1, keepdims=True)
    return (u - mu) / jnp.sqrt(var + 1e-5) * g + b


def _init_body(x_ref, w_ref, b_ref, a_ref, we_ref, be_ref,
               h_ref, hmax_ref, eamax_ref):
    i = pl.program_id(0)
    h = jnp.dot(x_ref[...], w_ref[...],
                preferred_element_type=jnp.float32) + b_ref[...]
    h_ref[...] = h

    @pl.when(i == 0)
    def _():
        hmax_ref[0, 0] = jnp.float32(NEG)
        aabs = jnp.maximum(jnp.max(a_ref[...]), -jnp.min(a_ref[...]))
        w = we_ref[...]
        ea = jnp.maximum(aabs * w, -aabs * w) + be_ref[...]
        eamax_ref[0, 0] = jnp.max(ea)

    hmax_ref[0, 0] = jnp.maximum(hmax_ref[0, 0], jnp.max(h))


def _init_call(x, node_W, node_b, a, edge_W, edge_b):
    smem = pl.BlockSpec((1, 1), lambda i: (0, 0), memory_space=pltpu.SMEM)
    return pl.pallas_call(
        _init_body,
        grid=(GRID,),
        in_specs=[
            pl.BlockSpec((R, D), lambda i: (i, 0)),
            pl.BlockSpec((D, H), lambda i: (0, 0)),
            pl.BlockSpec((1, H), lambda i: (0, 0)),
            pl.BlockSpec((E // 128, 128), lambda i: (0, 0)),
            pl.BlockSpec((1, H), lambda i: (0, 0)),
            pl.BlockSpec((1, H), lambda i: (0, 0)),
        ],
        out_specs=[pl.BlockSpec((R, H), lambda i: (i, 0)), smem, smem],
        out_shape=[
            jax.ShapeDtypeStruct((N, H), jnp.float32),
            jax.ShapeDtypeStruct((1, 1), jnp.float32),
            jax.ShapeDtypeStruct((1, 1), jnp.float32),
        ],
    )(x, node_W, node_b.reshape(1, H), a.reshape(-1, 128),
      edge_W, edge_b.reshape(1, H))


def _layer_body(has_res, h_ref, zin_ref, agg_ref, w1_ref, b1_ref, g1_ref,
                c1_ref, w2_ref, b2_ref, gn_ref, cn_ref,
                hn_ref, zn_ref, zmax_ref):
    i = pl.program_id(0)
    u = agg_ref[...] + zin_ref[...]
    u = jnp.dot(u, w1_ref[...], preferred_element_type=jnp.float32) + b1_ref[...]
    u = jnp.maximum(_ln(u, g1_ref[...], c1_ref[...]), 0.)
    u = jnp.dot(u, w2_ref[...], preferred_element_type=jnp.float32) + b2_ref[...]
    if has_res:
        hn = h_ref[...] + u
    else:
        hn = u
    hn_ref[...] = hn
    zn = jnp.maximum(_ln(hn, gn_ref[...], cn_ref[...]), 0.)
    zn_ref[...] = zn

    @pl.when(i == 0)
    def _():
        zmax_ref[0, 0] = jnp.float32(NEG)

    zmax_ref[0, 0] = jnp.maximum(zmax_ref[0, 0], jnp.max(zn))


def _layer_call(has_res, h, zin, agg, W1, b1, g1, c1, W2, b2, gn, cn):
    smem = pl.BlockSpec((1, 1), lambda i: (0, 0), memory_space=pltpu.SMEM)
    row = pl.BlockSpec((R, H), lambda i: (i, 0))
    const = lambda s: pl.BlockSpec(s, lambda i: (0, 0))
    return pl.pallas_call(
        functools.partial(_layer_body, has_res),
        grid=(GRID,),
        in_specs=[
            row, row, row,
            const((H, H2)), const((1, H2)), const((1, H2)), const((1, H2)),
            const((H2, H)), const((1, H)), const((1, H)), const((1, H)),
        ],
        out_specs=[row, row, smem],
        out_shape=[
            jax.ShapeDtypeStruct((N, H), jnp.float32),
            jax.ShapeDtypeStruct((N, H), jnp.float32),
            jax.ShapeDtypeStruct((1, 1), jnp.float32),
        ],
    )(h, zin, agg, W1, b1.reshape(1, H2), g1.reshape(1, H2),
      c1.reshape(1, H2), W2, b2.reshape(1, H), gn.reshape(1, H),
      cn.reshape(1, H))


def _head_body(z_ref, w_ref, b_ref, o_ref):
    logits = jnp.dot(z_ref[...], w_ref[...],
                     preferred_element_type=jnp.float32) + b_ref[...]
    mx = jnp.max(logits, axis=-1, keepdims=True)
    s = jnp.sum(jnp.exp(logits - mx), axis=-1, keepdims=True)
    o_ref[...] = logits - mx - jnp.log(s)


def _head_call(z, Wp, bp):
    return pl.pallas_call(
        _head_body,
        grid=(GRID,),
        in_specs=[
            pl.BlockSpec((R, H), lambda i: (i, 0)),
            pl.BlockSpec((H, 128), lambda i: (0, 0)),
            pl.BlockSpec((1, 128), lambda i: (0, 0)),
        ],
        out_specs=pl.BlockSpec((R, 128), lambda i: (i, 0)),
        out_shape=jax.ShapeDtypeStruct((N, 128), jnp.float32),
    )(z, Wp, bp)


def _conv_body(zT_hbm, eidx_hbm, par_hbm, agg_hbm,
               z_v, den_v, num_v, e_v, par_v):
    cid = lax.axis_index("c")
    sid = lax.axis_index("s")
    wid = sid * 2 + cid

    pltpu.sync_copy(zT_hbm.at[pl.ds(wid * (CPT * N), CPT * N)], z_v)
    pltpu.sync_copy(par_hbm.at[pl.ds(wid * PARW, PARW)], par_v)

    @plsc.parallel_loop(0, N // 16, 1, unroll=8)
    def zero_body(j):
        for c in range(CPT):
            den_v[pl.ds(c * N + j * 16, 16)] = jnp.zeros((16,), jnp.float32)
            num_v[pl.ds(c * N + j * 16, 16)] = jnp.zeros((16,), jnp.float32)

    wv = [par_v[pl.ds(c * 16, 16)] for c in range(CPT)]
    bv = [par_v[pl.ds((CPT + c) * 16, 16)] for c in range(CPT)]
    tv = par_v[pl.ds(2 * CPT * 16, 16)]
    Mv = par_v[pl.ds((2 * CPT + 1) * 16, 16)]

    cN = [jnp.full((16,), c * N, jnp.int32) for c in range(CPT)]

    def chunk_body(g, carry):
        pltpu.sync_copy(eidx_hbm.at[pl.ds(g * (3 * CHUNK), 3 * CHUNK)], e_v)

        @plsc.parallel_loop(0, CHUNK // 16, 1, unroll=6)
        def edge_body(j):
            s = e_v[pl.ds(j * 16, 16)]
            d = e_v[pl.ds(CHUNK + j * 16, 16)]
            av = plsc.bitcast(e_v[pl.ds(2 * CHUNK + j * 16, 16)],
                              jnp.float32)
            for c in range(CPT):
                zg = plsc.load_gather(z_v, [s + cN[c]])
                m = jnp.maximum(zg + av * wv[c] + bv[c], 0.) + EPS
                ex = jnp.exp(tv * m - Mv)
                dc = d + cN[c]
                plsc.addupdate_scatter(den_v, [dc], ex)
                plsc.addupdate_scatter(num_v, [dc], ex * m)

        return carry

    lax.fori_loop(0, NCHUNK, chunk_body, 0)

    @plsc.parallel_loop(0, N // 16, 1, unroll=4)
    def div_body(j):
        for c in range(CPT):
            sl = pl.ds(c * N + j * 16, 16)
            num_v[sl] = num_v[sl] / (den_v[sl] + 1e-16)

    pltpu.sync_copy(num_v, agg_hbm.at[pl.ds(wid * (CPT * N), CPT * N)])


@functools.cache
def _get_sc_conv():
    return pl.kernel(
        _conv_body,
        mesh=plsc.VectorSubcoreMesh(core_axis_name="c", subcore_axis_name="s"),
        out_type=jax.ShapeDtypeStruct((H * N,), jnp.float32),
        scratch_types=[
            pltpu.VMEM((CPT * N,), jnp.float32),
            pltpu.VMEM((CPT * N,), jnp.float32),
            pltpu.VMEM((CPT * N,), jnp.float32),
            pltpu.VMEM((3 * CHUNK,), jnp.int32),
            pltpu.VMEM((PARW,), jnp.float32),
        ],
        compiler_params=pltpu.CompilerParams(needs_layout_passes=False),
    )


def _sc_conv(zT, eidx, par):
    return _get_sc_conv()(zT, eidx, par)


def kernel(x, edge_index, edge_attr, node_W, node_b, edge_W, edge_b, t,
           mlp_W1, mlp_b1, mlp_ln_g, mlp_ln_b, mlp_W2, mlp_b2, ln_g, ln_b,
           lin_W, lin_b):
    a = edge_attr[:, 0]
    eidx = jnp.concatenate(
        [edge_index, lax.bitcast_convert_type(a, jnp.int32)[None]], axis=0)
    eidx = jnp.transpose(eidx.reshape(3, NCHUNK, CHUNK),
                         (1, 0, 2)).reshape(-1)

    h0, hmax, eamax = _init_call(x, node_W, node_b, a, edge_W, edge_b)
    hmax = hmax[0, 0]
    eamax = eamax[0, 0]

    we_r = edge_W[0].reshape(32, CPT)
    be_r = edge_b.reshape(32, CPT)

    h = h0
    z = h0
    zmax = hmax
    for i in range(L):
        msgmax = jnp.maximum(zmax + eamax, 0.) + EPS
        M = jnp.maximum(t[i] * msgmax, t[i] * EPS)
        par = jnp.concatenate(
            [we_r, be_r,
             jnp.full((32, 1), 1., jnp.float32) * t[i],
             jnp.full((32, 1), 1., jnp.float32) * M], axis=1)
        par = jnp.broadcast_to(par[:, :, None],
                               (32, 2 * CPT + 2, 16)).reshape(-1)
        aggT = _sc_conv(jnp.transpose(z).reshape(-1), eidx, par)
        agg = jnp.transpose(aggT.reshape(H, N))
        gn = ln_g[i + 1] if i < L - 1 else ln_g[0]
        cn = ln_b[i + 1] if i < L - 1 else ln_b[0]
        h, z, zmax = _layer_call(i > 0, h, z, agg, mlp_W1[i], mlp_b1[i],
                                 mlp_ln_g[i], mlp_ln_b[i], mlp_W2[i],
                                 mlp_b2[i], gn, cn)
        zmax = zmax[0, 0]

    Wp = jnp.pad(lin_W, ((0, 0), (0, 128 - C)))
    bp = jnp.pad(lin_b, (0, 128 - C), constant_values=NEG).reshape(1, 128)
    out = _head_call(z, Wp, bp)
    return out[:, :C]


# speedup vs baseline: 3.5566x; 1.0125x over previous
import functools

import jax
import jax.numpy as jnp
from jax import lax
from jax.experimental import pallas as pl
from jax.experimental.pallas import tpu as pltpu
from jax.experimental.pallas import tpu_sc as plsc

N = 10000
E = 320000
D = 128
H = 128
H2 = 256
L = 7
C = 40
EPS = 1e-7

R = 1000
GRID = N // R
CHUNK = 2560
NCHUNK = E // CHUNK
CPT = 4
PARW = (2 * CPT + 2) * 16
NEG = -1e30


def _ln(u, g, b):
    mu = jnp.mean(u, axis=-1, keepdims=True)
    var = jnp.var(u, axis=-1, keepdims=True)
    return (u - mu) / jnp.sqrt(var + 1e-5) * g + b


def _init_body(x_ref, w_ref, b_ref, a_ref, we_ref, be_ref,
               h_ref, hmax_ref, eamax_ref):
    i = pl.program_id(0)
    h = jnp.dot(x_ref[...], w_ref[...],
                preferred_element_type=jnp.float32) + b_ref[...]
    h_ref[...] = h

    @pl.when(i == 0)
    def _():
        hmax_ref[0, 0] = jnp.float32(NEG)
        aabs = jnp.maximum(jnp.max(a_ref[...]), -jnp.min(a_ref[...]))
        w = we_ref[...]
        ea = jnp.maximum(aabs * w, -aabs * w) + be_ref[...]
        eamax_ref[0, 0] = jnp.max(ea)

    hmax_ref[0, 0] = jnp.maximum(hmax_ref[0, 0], jnp.max(h))


def _init_call(x, node_W, node_b, a, edge_W, edge_b):
    smem = pl.BlockSpec((1, 1), lambda i: (0, 0), memory_space=pltpu.SMEM)
    return pl.pallas_call(
        _init_body,
        grid=(GRID,),
        in_specs=[
            pl.BlockSpec((R, D), lambda i: (i, 0)),
            pl.BlockSpec((D, H), lambda i: (0, 0)),
            pl.BlockSpec((1, H), lambda i: (0, 0)),
            pl.BlockSpec((E // 128, 128), lambda i: (0, 0)),
            pl.BlockSpec((1, H), lambda i: (0, 0)),
            pl.BlockSpec((1, H), lambda i: (0, 0)),
        ],
        out_specs=[pl.BlockSpec((R, H), lambda i: (i, 0)), smem, smem],
        out_shape=[
            jax.ShapeDtypeStruct((N, H), jnp.float32),
            jax.ShapeDtypeStruct((1, 1), jnp.float32),
            jax.ShapeDtypeStruct((1, 1), jnp.float32),
        ],
    )(x, node_W, node_b.reshape(1, H), a.reshape(-1, 128),
      edge_W, edge_b.reshape(1, H))


def _lnT(u, g, b):
    mu = jnp.mean(u, axis=0, keepdims=True)
    var = jnp.var(u, axis=0, keepdims=True)
    return (u - mu) / jnp.sqrt(var + 1e-5) * g + b


_DN00 = (((0,), (0,)), ((), ()))


def _layer_body(has_res, h_ref, zin_ref, agg_ref, w1_ref, b1_ref, g1_ref,
                c1_ref, w2_ref, b2_ref, gn_ref, cn_ref,
                hn_ref, zn_ref, zmax_ref):
    i = pl.program_id(0)
    u = agg_ref[...] + zin_ref[...]
    u = lax.dot_general(w1_ref[...], u, _DN00,
                        preferred_element_type=jnp.float32) + b1_ref[...]
    u = jnp.maximum(_lnT(u, g1_ref[...], c1_ref[...]), 0.)
    u = lax.dot_general(w2_ref[...], u, _DN00,
                        preferred_element_type=jnp.float32) + b2_ref[...]
    if has_res:
        hn = h_ref[...] + u
    else:
        hn = u
    hn_ref[...] = hn
    zn = jnp.maximum(_lnT(hn, gn_ref[...], cn_ref[...]), 0.)
    zn_ref[...] = zn

    @pl.when(i == 0)
    def _():
        zmax_ref[0, 0] = jnp.float32(NEG)

    zmax_ref[0, 0] = jnp.maximum(zmax_ref[0, 0], jnp.max(zn))


def _layer_call(has_res, hT, zinT, aggT, W1, b1, g1, c1, W2, b2, gn, cn):
    smem = pl.BlockSpec((1, 1), lambda i: (0, 0), memory_space=pltpu.SMEM)
    col = pl.BlockSpec((H, N), lambda i: (0, 0))
    const = lambda s: pl.BlockSpec(s, lambda i: (0, 0))
    return pl.pallas_call(
        functools.partial(_layer_body, has_res),
        grid=(1,),
        in_specs=[
            col, col, col,
            const((H, H2)), const((H2, 1)), const((H2, 1)), const((H2, 1)),
            const((H2, H)), const((H, 1)), const((H, 1)), const((H, 1)),
        ],
        out_specs=[col, col, smem],
        out_shape=[
            jax.ShapeDtypeStruct((H, N), jnp.float32),
            jax.ShapeDtypeStruct((H, N), jnp.float32),
            jax.ShapeDtypeStruct((1, 1), jnp.float32),
        ],
    )(hT, zinT, aggT, W1, b1.reshape(H2, 1), g1.reshape(H2, 1),
      c1.reshape(H2, 1), W2, b2.reshape(H, 1), gn.reshape(H, 1),
      cn.reshape(H, 1))


def _head_body(z_ref, w_ref, b_ref, o_ref):
    logits = lax.dot_general(z_ref[...], w_ref[...], _DN00,
                             preferred_element_type=jnp.float32) + b_ref[...]
    mx = jnp.max(logits, axis=-1, keepdims=True)
    s = jnp.sum(jnp.exp(logits - mx), axis=-1, keepdims=True)
    o_ref[...] = logits - mx - jnp.log(s)


def _head_call(zT, Wp, bp):
    return pl.pallas_call(
        _head_body,
        grid=(1,),
        in_specs=[
            pl.BlockSpec((H, N), lambda i: (0, 0)),
            pl.BlockSpec((H, 128), lambda i: (0, 0)),
            pl.BlockSpec((1, 128), lambda i: (0, 0)),
        ],
        out_specs=pl.BlockSpec((N, 128), lambda i: (0, 0)),
        out_shape=jax.ShapeDtypeStruct((N, 128), jnp.float32),
    )(zT, Wp, bp)


def _conv_body(zT_hbm, eidx_hbm, par_hbm, agg_hbm,
               z_v, den_v, num_v, e_v, par_v):
    cid = lax.axis_index("c")
    sid = lax.axis_index("s")
    wid = sid * 2 + cid

    pltpu.sync_copy(zT_hbm.at[pl.ds(wid * (CPT * N), CPT * N)], z_v)
    pltpu.sync_copy(par_hbm.at[pl.ds(wid * PARW, PARW)], par_v)

    @plsc.parallel_loop(0, N // 16, 1, unroll=8)
    def zero_body(j):
        for c in range(CPT):
            den_v[pl.ds(c * N + j * 16, 16)] = jnp.zeros((16,), jnp.float32)
            num_v[pl.ds(c * N + j * 16, 16)] = jnp.zeros((16,), jnp.float32)

    wv = [par_v[pl.ds(c * 16, 16)] for c in range(CPT)]
    bv = [par_v[pl.ds((CPT + c) * 16, 16)] for c in range(CPT)]
    tv = par_v[pl.ds(2 * CPT * 16, 16)]
    Mv = par_v[pl.ds((2 * CPT + 1) * 16, 16)]

    cN = [jnp.full((16,), c * N, jnp.int32) for c in range(CPT)]

    def chunk_body(g, carry):
        pltpu.sync_copy(eidx_hbm.at[pl.ds(g * (3 * CHUNK), 3 * CHUNK)], e_v)

        @plsc.parallel_loop(0, CHUNK // 16, 1, unroll=6)
        def edge_body(j):
            s = e_v[pl.ds(j * 16, 16)]
            d = e_v[pl.ds(CHUNK + j * 16, 16)]
            av = plsc.bitcast(e_v[pl.ds(2 * CHUNK + j * 16, 16)],
                              jnp.float32)
            for c in range(CPT):
                zg = plsc.load_gather(z_v, [s + cN[c]])
                m = jnp.maximum(zg + av * wv[c] + bv[c], 0.) + EPS
                ex = jnp.exp(tv * m - Mv)
                dc = d + cN[c]
                plsc.addupdate_scatter(den_v, [dc], ex)
                plsc.addupdate_scatter(num_v, [dc], ex * m)

        return carry

    lax.fori_loop(0, NCHUNK, chunk_body, 0)

    @plsc.parallel_loop(0, N // 16, 1, unroll=4)
    def div_body(j):
        for c in range(CPT):
            sl = pl.ds(c * N + j * 16, 16)
            num_v[sl] = num_v[sl] / (den_v[sl] + 1e-16)

    pltpu.sync_copy(num_v, agg_hbm.at[pl.ds(wid * (CPT * N), CPT * N)])


@functools.cache
def _get_sc_conv():
    return pl.kernel(
        _conv_body,
        mesh=plsc.VectorSubcoreMesh(core_axis_name="c", subcore_axis_name="s"),
        out_type=jax.ShapeDtypeStruct((H * N,), jnp.float32),
        scratch_types=[
            pltpu.VMEM((CPT * N,), jnp.float32),
            pltpu.VMEM((CPT * N,), jnp.float32),
            pltpu.VMEM((CPT * N,), jnp.float32),
            pltpu.VMEM((3 * CHUNK,), jnp.int32),
            pltpu.VMEM((PARW,), jnp.float32),
        ],
        compiler_params=pltpu.CompilerParams(needs_layout_passes=False),
    )


def _sc_conv(zT, eidx, par):
    return _get_sc_conv()(zT, eidx, par)


def kernel(x, edge_index, edge_attr, node_W, node_b, edge_W, edge_b, t,
           mlp_W1, mlp_b1, mlp_ln_g, mlp_ln_b, mlp_W2, mlp_b2, ln_g, ln_b,
           lin_W, lin_b):
    a = edge_attr[:, 0]
    eidx = jnp.concatenate(
        [edge_index, lax.bitcast_convert_type(a, jnp.int32)[None]], axis=0)
    eidx = jnp.transpose(eidx.reshape(3, NCHUNK, CHUNK),
                         (1, 0, 2)).reshape(-1)

    h0, hmax, eamax = _init_call(x, node_W, node_b, a, edge_W, edge_b)
    hmax = hmax[0, 0]
    eamax = eamax[0, 0]

    we_r = edge_W[0].reshape(32, CPT)
    be_r = edge_b.reshape(32, CPT)

    h0T = jnp.transpose(h0)
    hT = h0T
    zT = h0T
    zmax = hmax
    for i in range(L):
        msgmax = jnp.maximum(zmax + eamax, 0.) + EPS
        M = jnp.maximum(t[i] * msgmax, t[i] * EPS)
        par = jnp.concatenate(
            [we_r, be_r,
             jnp.full((32, 1), 1., jnp.float32) * t[i],
             jnp.full((32, 1), 1., jnp.float32) * M], axis=1)
        par = jnp.broadcast_to(par[:, :, None],
                               (32, 2 * CPT + 2, 16)).reshape(-1)
        aggT = _sc_conv(zT.reshape(-1), eidx, par).reshape(H, N)
        gn = ln_g[i + 1] if i < L - 1 else ln_g[0]
        cn = ln_b[i + 1] if i < L - 1 else ln_b[0]
        hT, zT, zmax = _layer_call(i > 0, hT, zT, aggT, mlp_W1[i],
                                   mlp_b1[i], mlp_ln_g[i], mlp_ln_b[i],
                                   mlp_W2[i], mlp_b2[i], gn, cn)
        zmax = zmax[0, 0]

    Wp = jnp.pad(lin_W, ((0, 0), (0, 128 - C)))
    bp = jnp.pad(lin_b, (0, 128 - C), constant_values=NEG).reshape(1, 128)
    out = _head_call(zT, Wp, bp)
    return out[:, :C]
